# Initial kernel scaffold; baseline (speedup 1.0000x reference)
import jax
import jax.numpy as jnp
from jax.experimental import pallas as pl


def kernel(indices, table):
    raise NotImplementedError("write your pallas kernel here")


# speedup vs baseline: 2.8669x; 2.8669x over previous
import functools

import jax
import jax.numpy as jnp
from jax import lax
from jax.experimental import pallas as pl
from jax.experimental.pallas import tpu as pltpu
from jax.experimental.pallas import tpu_sc as plsc

_VOCAB = 100000
_D = 50
_DP = 56
_B = 16384 * 50

_info = plsc.get_sparse_core_info()
_NC, _NS = _info.num_cores, _info.num_subcores
_NW = _NC * _NS
_BPW = _B // _NW
_G = 128
_NG = _BPW // _G

_mesh = plsc.VectorSubcoreMesh(core_axis_name="c", subcore_axis_name="s")


@functools.partial(
    pl.kernel,
    out_type=jax.ShapeDtypeStruct((_B, _DP), jnp.float32),
    mesh=_mesh,
    scratch_types=[
        pltpu.VMEM((_G,), jnp.int32),
        pltpu.VMEM((_G, _DP), jnp.float32),
        pltpu.SemaphoreType.DMA,
    ],
    compiler_params=pltpu.CompilerParams(use_tc_tiling_on_sc=False),
)
def _gather(idx_hbm, table_hbm, out_hbm, idx_g, rows_v, sem):
    wid = lax.axis_index("s") * _NC + lax.axis_index("c")
    base = wid * _BPW

    def group(j, carry):
        pltpu.sync_copy(idx_hbm.at[wid * _NG + j], idx_g)
        pltpu.async_copy(table_hbm.at[idx_g], rows_v, sem).wait()
        pltpu.sync_copy(rows_v, out_hbm.at[pl.ds(base + j * _G, _G)])
        return carry

    lax.fori_loop(0, _NG, group, 0)


def kernel(indices, table):
    idx2 = indices.reshape(_NW * _NG, _G).astype(jnp.int32)
    table_p = jnp.pad(table, ((0, 0), (0, _DP - _D)))
    out = _gather(idx2, table_p)
    return out[:, :_D].reshape(indices.shape + (table.shape[1],))


# speedup vs baseline: 3.4487x; 1.2029x over previous
import functools

import jax
import jax.numpy as jnp
from jax import lax
from jax.experimental import pallas as pl
from jax.experimental.pallas import tpu as pltpu
from jax.experimental.pallas import tpu_sc as plsc

_VOCAB = 100000
_D = 50
_DP = 56
_B = 16384 * 50

_info = plsc.get_sparse_core_info()
_NC, _NS = _info.num_cores, _info.num_subcores
_NW = _NC * _NS
_BPW = _B // _NW
_G = 512
_NG = _BPW // _G

_mesh = plsc.VectorSubcoreMesh(core_axis_name="c", subcore_axis_name="s")


@functools.partial(
    pl.kernel,
    out_type=jax.ShapeDtypeStruct((_B, _DP), jnp.float32),
    mesh=_mesh,
    scratch_types=[
        pltpu.VMEM((_NG, _G), jnp.int32),
        pltpu.VMEM((2, _G, _DP), jnp.float32),
        pltpu.SemaphoreType.DMA,
        pltpu.SemaphoreType.DMA,
    ],
    compiler_params=pltpu.CompilerParams(use_tc_tiling_on_sc=False),
)
def _gather(idx_hbm, table_hbm, out_hbm, idx_v, rows_v, sem0, sem1):
    wid = lax.axis_index("s") * _NC + lax.axis_index("c")
    base = wid * _BPW
    sems = (sem0, sem1)

    pltpu.sync_copy(idx_hbm.at[pl.ds(wid * _NG, _NG)], idx_v)
    pltpu.async_copy(table_hbm.at[idx_v.at[0]], rows_v.at[0], sems[0])

    def outer(j2, carry):
        for b in range(2):
            j = j2 * 2 + b
            nb = 1 - b

            @pl.when(j + 1 < _NG)
            def _fire():
                pltpu.async_copy(
                    table_hbm.at[idx_v.at[j + 1]], rows_v.at[nb], sems[nb]
                )

            pltpu.make_async_copy(
                table_hbm.at[idx_v.at[j]], rows_v.at[b], sems[b]
            ).wait()
            pltpu.sync_copy(rows_v.at[b], out_hbm.at[pl.ds(base + j * _G, _G)])
        return carry

    lax.fori_loop(0, _NG // 2, outer, 0)


def kernel(indices, table):
    idx2 = indices.reshape(_NW * _NG, _G).astype(jnp.int32)
    table_p = jnp.pad(table, ((0, 0), (0, _DP - _D)))
    out = _gather(idx2, table_p)
    return out[:, :_D].reshape(indices.shape + (table.shape[1],))


# speedup vs baseline: 5.2301x; 1.5166x over previous
import functools

import jax
import jax.numpy as jnp
from jax import lax
from jax.experimental import pallas as pl
from jax.experimental.pallas import tpu as pltpu
from jax.experimental.pallas import tpu_sc as plsc

_VOCAB = 100000
_D = 50
_DP = 56
_BATCH = 16384
_B = _BATCH * _D

_info = plsc.get_sparse_core_info()
_NC, _NS = _info.num_cores, _info.num_subcores
_NW = _NC * _NS
_BPW = _B // _NW
_RPW = _BPW // _D
_NB = 8
_G = _NB * _D
_NG = _BPW // _G
_L = 16

_mesh = plsc.VectorSubcoreMesh(core_axis_name="c", subcore_axis_name="s")


@functools.partial(
    pl.kernel,
    out_type=jax.ShapeDtypeStruct((_BATCH, _D, _DP), jnp.float32),
    mesh=_mesh,
    scratch_types=[
        pltpu.VMEM((_RPW, _D), jnp.int32),
        pltpu.VMEM((_NG, _G), jnp.int32),
        pltpu.VMEM((2, _G, _DP), jnp.float32),
        pltpu.SemaphoreType.DMA,
        pltpu.SemaphoreType.DMA,
        pltpu.SemaphoreType.DMA,
        pltpu.SemaphoreType.DMA,
    ],
    compiler_params=pltpu.CompilerParams(
        use_tc_tiling_on_sc=False, needs_layout_passes=False
    ),
)
def _gather(idx_hbm, table_hbm, out_hbm, idx_raw, idx_c, rows_v,
            gsem0, gsem1, osem0, osem1):
    wid = lax.axis_index("s") * _NC + lax.axis_index("c")
    ebase = wid * _RPW
    gsems = (gsem0, gsem1)
    osems = (osem0, osem1)

    pltpu.sync_copy(idx_hbm.at[pl.ds(ebase, _RPW)], idx_raw)

    lane = lax.iota(jnp.int32, _L)

    def compact(k, carry):
        e = k * _L + lane
        vals = plsc.load_gather(idx_raw, [e // _D, e % _D])
        idx_c[(k * _L) // _G, pl.ds((k * _L) % _G, _L)] = vals
        return carry

    lax.fori_loop(0, _BPW // _L, compact, 0)

    def out_copies(buf, j, fn):
        for m in range(_NB):
            fn(
                rows_v.at[buf].at[pl.ds(m * _D, _D)],
                out_hbm.at[ebase + j * _NB + m],
                osems[buf],
            )

    pltpu.async_copy(table_hbm.at[idx_c.at[0]], rows_v.at[0], gsems[0])

    def outer(j2, carry):
        for b in range(2):
            j = j2 * 2 + b
            nb = 1 - b

            @pl.when(j + 1 < _NG)
            def _fire():
                @pl.when(j >= 1)
                def _drain():
                    out_copies(
                        nb, j - 1,
                        lambda s, d, sem: pltpu.make_async_copy(s, d, sem).wait(),
                    )

                pltpu.async_copy(
                    table_hbm.at[idx_c.at[j + 1]], rows_v.at[nb], gsems[nb]
                )

            pltpu.make_async_copy(
                table_hbm.at[idx_c.at[j]], rows_v.at[b], gsems[b]
            ).wait()
            out_copies(b, j, pltpu.async_copy)
        return carry

    lax.fori_loop(0, _NG // 2, outer, 0)

    out_copies(0, _NG - 2,
               lambda s, d, sem: pltpu.make_async_copy(s, d, sem).wait())
    out_copies(1, _NG - 1,
               lambda s, d, sem: pltpu.make_async_copy(s, d, sem).wait())


def kernel(indices, table):
    table_p = jnp.pad(table, ((0, 0), (0, _DP - _D)))
    out = _gather(indices.astype(jnp.int32), table_p)
    return out[:, :, :_D]
